# baseline (speedup 1.0000x reference)
import functools

import jax
import jax.numpy as jnp
from jax import lax
from jax.experimental import pallas as pl
from jax.experimental.pallas import tpu as pltpu
from jax.experimental.pallas import tpu_sc as plsc

N, E, D, IN = 10000, 320000, 128, 1024
D2 = 2 * D
NTILES = 32
CH = 128
TCH = 2528
NCH0 = 66
NCH1 = 92
EPAD = TCH * CH
DUMMY = N
NP = N + 16
RPT = 624
RC = 104
RTAIL = N - 16 * RPT
BM = 400
NBLK = N // BM

_f32 = jnp.float32


def _sc_body(with_ea, *refs):
    if with_ea:
        (h_hbm, src_hbm, dst_hbm, eid_hbm, ea_hbm, out_h, out_ea,
         src_v, dst_v, rows_v, sem, acc_h, eid_v, ea_v, acc_ea) = refs
    else:
        (h_hbm, src_hbm, dst_hbm, out_h,
         src_v, dst_v, rows_v, sem, acc_h) = refs

    cid = lax.axis_index("c")
    sid = lax.axis_index("s")
    base = jnp.where(cid == 0, sid * NCH0, 16 * NCH0 + sid * NCH1)
    nw = jnp.where(cid == 0, NCH0, NCH1)

    for k in range(RPT // RC):
        r0 = sid * RPT + k * RC
        pltpu.sync_copy(h_hbm.at[pl.ds(r0, RC)], rows_v.at[pl.ds(0, RC)])
        pltpu.sync_copy(rows_v.at[pl.ds(0, RC)], acc_h.at[pl.ds(r0, RC)])

    @pl.when(sid == 15)
    def _():
        pltpu.sync_copy(h_hbm.at[pl.ds(16 * RPT, RTAIL)],
                        rows_v.at[pl.ds(0, RTAIL)])
        pltpu.sync_copy(rows_v.at[pl.ds(0, RTAIL)],
                        acc_h.at[pl.ds(16 * RPT, RTAIL)])

    if with_ea:
        def zrow(j, carry):
            ea_v[j, :] = jnp.zeros((16,), _f32)
            return carry

        lax.fori_loop(0, CH, zrow, 0)
        for k in range(RPT // RC):
            r0 = sid * RPT + k * RC
            pltpu.sync_copy(ea_v.at[pl.ds(0, RC)], acc_ea.at[pl.ds(r0, RC)])

        @pl.when(sid == 15)
        def _():
            pltpu.sync_copy(ea_v.at[pl.ds(0, RTAIL)],
                            acc_ea.at[pl.ds(16 * RPT, RTAIL)])

    plsc.subcore_barrier()

    def chunk(j, carry):
        q = base + j
        pltpu.sync_copy(src_hbm.at[q], src_v)
        pltpu.sync_copy(dst_hbm.at[q], dst_v)
        pltpu.async_copy(h_hbm.at[src_v], rows_v, sem).wait()
        pltpu.sync_copy(rows_v, acc_h.at[dst_v], add=True)
        if with_ea:
            pltpu.sync_copy(eid_hbm.at[q], eid_v)
            pltpu.async_copy(ea_hbm.at[eid_v], ea_v, sem).wait()
            pltpu.sync_copy(ea_v, acc_ea.at[dst_v], add=True)
        return carry

    lax.fori_loop(0, nw, chunk, 0)
    plsc.subcore_barrier()

    for k in range(RPT // RC):
        r0 = sid * RPT + k * RC
        pltpu.sync_copy(acc_h.at[pl.ds(r0, RC)], rows_v.at[pl.ds(0, RC)])
        pltpu.sync_copy(rows_v.at[pl.ds(0, RC)],
                        out_h.at[cid, pl.ds(r0, RC)])
        if with_ea:
            pltpu.sync_copy(acc_ea.at[pl.ds(r0, RC)], ea_v.at[pl.ds(0, RC)])
            pltpu.sync_copy(ea_v.at[pl.ds(0, RC)],
                            out_ea.at[cid, pl.ds(r0, RC)])

    @pl.when(sid == 15)
    def _():
        pltpu.sync_copy(acc_h.at[pl.ds(16 * RPT, RTAIL)],
                        rows_v.at[pl.ds(0, RTAIL)])
        pltpu.sync_copy(rows_v.at[pl.ds(0, RTAIL)],
                        out_h.at[cid, pl.ds(16 * RPT, RTAIL)])
        if with_ea:
            pltpu.sync_copy(acc_ea.at[pl.ds(16 * RPT, RTAIL)],
                            ea_v.at[pl.ds(0, RTAIL)])
            pltpu.sync_copy(ea_v.at[pl.ds(0, RTAIL)],
                            out_ea.at[cid, pl.ds(16 * RPT, RTAIL)])


def _make_sc(with_ea):
    mesh = plsc.VectorSubcoreMesh(core_axis_name="c", subcore_axis_name="s")
    out_type = [jax.ShapeDtypeStruct((2, N, D), _f32)]
    scratch = [
        pltpu.VMEM((CH,), jnp.int32),
        pltpu.VMEM((CH,), jnp.int32),
        pltpu.VMEM((CH, D), _f32),
        pltpu.SemaphoreType.DMA,
        pltpu.VMEM_SHARED((NP, D), _f32),
    ]
    if with_ea:
        out_type.append(jax.ShapeDtypeStruct((2, N, 16), _f32))
        scratch += [
            pltpu.VMEM((CH,), jnp.int32),
            pltpu.VMEM((CH, 16), _f32),
            pltpu.VMEM_SHARED((NP, 16), _f32),
        ]
    return pl.kernel(functools.partial(_sc_body, with_ea),
                     out_type=out_type, mesh=mesh, scratch_types=scratch,
                     compiler_params=pltpu.CompilerParams(
                         use_tc_tiling_on_sc=False))


_sc_l0 = _make_sc(True)
_sc_l1 = _make_sc(False)


def _proj_body(x_ref, w_ref, b_ref, o_ref):
    o_ref[...] = jnp.dot(x_ref[...], w_ref[...],
                         preferred_element_type=_f32) + b_ref[...]


def _proj(x, wT, b):
    return pl.pallas_call(
        _proj_body,
        grid=(NBLK,),
        in_specs=[pl.BlockSpec((BM, IN), lambda i: (i, 0)),
                  pl.BlockSpec((IN, D), lambda i: (0, 0)),
                  pl.BlockSpec((1, D), lambda i: (0, 0))],
        out_specs=pl.BlockSpec((BM, D), lambda i: (i, 0)),
        out_shape=jax.ShapeDtypeStruct((N, D), _f32),
    )(x, wT, b)


def _pre_body(p0, p1, h_ref, e0, e1, w1, bf, br, z_ref, s_ref, q_ref):
    a = p0[0] + p1[0] - h_ref[...]
    e = e0[0] + e1[0]
    z = (jnp.dot(a, w1[...], preferred_element_type=_f32)
         + jnp.dot(e, bf[...], preferred_element_type=_f32) + br[...])
    z_ref[...] = z

    @pl.when(pl.program_id(0) == 0)
    def _():
        s_ref[...] = jnp.zeros_like(s_ref)
        q_ref[...] = jnp.zeros_like(q_ref)

    s_ref[...] += jnp.sum(z, axis=0, keepdims=True)
    q_ref[...] += jnp.sum(z * z, axis=0, keepdims=True)


def _pre(ph, h, pe, w1l, bfull, brow):
    return pl.pallas_call(
        _pre_body,
        grid=(NBLK,),
        in_specs=[pl.BlockSpec((1, BM, D), lambda i: (0, i, 0)),
                  pl.BlockSpec((1, BM, D), lambda i: (1, i, 0)),
                  pl.BlockSpec((BM, D), lambda i: (i, 0)),
                  pl.BlockSpec((1, BM, 16), lambda i: (0, i, 0)),
                  pl.BlockSpec((1, BM, 16), lambda i: (1, i, 0)),
                  pl.BlockSpec((D, D2), lambda i: (0, 0)),
                  pl.BlockSpec((16, D2), lambda i: (0, 0)),
                  pl.BlockSpec((1, D2), lambda i: (0, 0))],
        out_specs=[pl.BlockSpec((BM, D2), lambda i: (i, 0)),
                   pl.BlockSpec((1, D2), lambda i: (0, 0)),
                   pl.BlockSpec((1, D2), lambda i: (0, 0))],
        out_shape=[jax.ShapeDtypeStruct((N, D2), _f32),
                   jax.ShapeDtypeStruct((1, D2), _f32),
                   jax.ShapeDtypeStruct((1, D2), _f32)],
    )(ph, ph, h, pe, pe, w1l, bfull, brow)


def _post_body(outer_relu, z_ref, s_ref, q_ref, g_ref, bt_ref, w2_ref,
               b2_ref, o_ref):
    mu = s_ref[...] / N
    var = q_ref[...] / N - mu * mu
    inv = lax.rsqrt(var + 1e-5)
    zn = (z_ref[...] - mu) * inv * g_ref[...] + bt_ref[...]
    zr = jnp.maximum(zn, 0.0)
    o = jnp.dot(zr, w2_ref[...], preferred_element_type=_f32) + b2_ref[...]
    if outer_relu:
        o = jnp.maximum(o, 0.0)
    o_ref[...] = o


def _post(z, s, q, g, bt, w2T, b2, outer_relu):
    return pl.pallas_call(
        functools.partial(_post_body, outer_relu),
        grid=(NBLK,),
        in_specs=[pl.BlockSpec((BM, D2), lambda i: (i, 0)),
                  pl.BlockSpec((1, D2), lambda i: (0, 0)),
                  pl.BlockSpec((1, D2), lambda i: (0, 0)),
                  pl.BlockSpec((1, D2), lambda i: (0, 0)),
                  pl.BlockSpec((1, D2), lambda i: (0, 0)),
                  pl.BlockSpec((D2, D), lambda i: (0, 0)),
                  pl.BlockSpec((1, D), lambda i: (0, 0))],
        out_specs=pl.BlockSpec((BM, D), lambda i: (i, 0)),
        out_shape=jax.ShapeDtypeStruct((N, D), _f32),
    )(z, s, q, g, bt, w2T, b2)


def _layer(h, ph, pe, W1, b1, We, be, g, bt, W2, b2, outer_relu):
    w1l = W1[:, :D].T
    w1rT = W1[:, D:].T
    b7 = We.T @ w1rT
    c = be @ w1rT
    bfull = jnp.zeros((16, D2), _f32).at[:7].set(b7).at[7].set(c)
    brow = (b1 + b7[6] + c).reshape(1, D2)
    z, s, q = _pre(ph, h, pe, w1l, bfull, brow)
    return _post(z, s, q, g.reshape(1, D2), bt.reshape(1, D2), W2.T,
                 b2.reshape(1, D), outer_relu)


def kernel(x, edge_index, edge_attr, Win, bin_, We0, be0, W1_0, b1_0, g0,
           bt0, W2_0, b2_0, We1, be1, W1_1, b1_1, g1, bt1, W2_1, b2_1):
    src = edge_index[0].astype(jnp.int32)
    dst = edge_index[1].astype(jnp.int32)
    src_r = jnp.pad(src, (0, EPAD - E)).reshape(TCH, CH)
    dst_r = jnp.pad(dst, (0, EPAD - E),
                    constant_values=DUMMY).reshape(TCH, CH)
    ea = jnp.concatenate(
        [edge_attr.astype(_f32), jnp.ones((E, 1), _f32),
         jnp.zeros((E, 8), _f32)], axis=1)
    ea_r = jnp.pad(ea, ((0, EPAD - E), (0, 0)))
    eid_r = jnp.arange(EPAD, dtype=jnp.int32).reshape(TCH, CH)

    h0 = _proj(x, Win.T, bin_.reshape(1, D))

    ph0, pe = _sc_l0(h0, src_r, dst_r, eid_r, ea_r)
    h1 = _layer(h0, ph0, pe, W1_0, b1_0, We0, be0, g0, bt0, W2_0, b2_0,
                outer_relu=True)

    (ph1,) = _sc_l1(h1, src_r, dst_r)
    return _layer(h1, ph1, pe, W1_1, b1_1, We1, be1, g1, bt1, W2_1, b2_1,
                  outer_relu=False)


# speedup vs baseline: 1.1292x; 1.1292x over previous
import functools

import jax
import jax.numpy as jnp
from jax import lax
from jax.experimental import pallas as pl
from jax.experimental.pallas import tpu as pltpu
from jax.experimental.pallas import tpu_sc as plsc

N, E, D, IN = 10000, 320000, 128, 1024
D2 = 2 * D
NTILES = 32
CH = 128
TCH = 2528
NCH0 = 92
NCH1 = 66
EPAD = TCH * CH
DUMMY = N
NP = N + 16
RPT = 624
RC = 104
RTAIL = N - 16 * RPT
BM = 400
NBLK = N // BM

_f32 = jnp.float32


def _sc_body(with_ea, *refs):
    if with_ea:
        (h_hbm, src_hbm, dst_hbm, eid_hbm, ea_hbm, out_h, out_ea,
         src_v, dst_v, rows_v, sem, acc_h, eid_v, ea_v, acc_ea) = refs
    else:
        (h_hbm, src_hbm, dst_hbm, out_h,
         src_v, dst_v, rows_v, sem, acc_h) = refs

    cid = lax.axis_index("c")
    sid = lax.axis_index("s")
    base = jnp.where(cid == 0, sid * NCH0, 16 * NCH0 + sid * NCH1)
    nw = jnp.where(cid == 0, NCH0, NCH1)

    for k in range(RPT // RC):
        r0 = sid * RPT + k * RC
        pltpu.sync_copy(h_hbm.at[pl.ds(r0, RC)], rows_v.at[pl.ds(0, RC)])
        pltpu.sync_copy(rows_v.at[pl.ds(0, RC)], acc_h.at[pl.ds(r0, RC)])

    @pl.when(sid == 15)
    def _():
        pltpu.sync_copy(h_hbm.at[pl.ds(16 * RPT, RTAIL)],
                        rows_v.at[pl.ds(0, RTAIL)])
        pltpu.sync_copy(rows_v.at[pl.ds(0, RTAIL)],
                        acc_h.at[pl.ds(16 * RPT, RTAIL)])

    if with_ea:
        def zrow(j, carry):
            ea_v[j, :] = jnp.zeros((16,), _f32)
            return carry

        lax.fori_loop(0, CH, zrow, 0)
        for k in range(RPT // RC):
            r0 = sid * RPT + k * RC
            pltpu.sync_copy(ea_v.at[pl.ds(0, RC)], acc_ea.at[pl.ds(r0, RC)])

        @pl.when(sid == 15)
        def _():
            pltpu.sync_copy(ea_v.at[pl.ds(0, RTAIL)],
                            acc_ea.at[pl.ds(16 * RPT, RTAIL)])

    plsc.subcore_barrier()

    def chunk(j, carry):
        q = base + j
        pltpu.sync_copy(src_hbm.at[q], src_v)
        pltpu.sync_copy(dst_hbm.at[q], dst_v)
        pltpu.async_copy(h_hbm.at[src_v], rows_v, sem).wait()
        pltpu.sync_copy(rows_v, acc_h.at[dst_v], add=True)
        if with_ea:
            pltpu.sync_copy(eid_hbm.at[q], eid_v)
            pltpu.async_copy(ea_hbm.at[eid_v], ea_v, sem).wait()
            pltpu.sync_copy(ea_v, acc_ea.at[dst_v], add=True)
        return carry

    lax.fori_loop(0, nw, chunk, 0)
    plsc.subcore_barrier()

    for k in range(RPT // RC):
        r0 = sid * RPT + k * RC
        pltpu.sync_copy(acc_h.at[pl.ds(r0, RC)], rows_v.at[pl.ds(0, RC)])
        pltpu.sync_copy(rows_v.at[pl.ds(0, RC)],
                        out_h.at[cid, pl.ds(r0, RC)])
        if with_ea:
            pltpu.sync_copy(acc_ea.at[pl.ds(r0, RC)], ea_v.at[pl.ds(0, RC)])
            pltpu.sync_copy(ea_v.at[pl.ds(0, RC)],
                            out_ea.at[cid, pl.ds(r0, RC)])

    @pl.when(sid == 15)
    def _():
        pltpu.sync_copy(acc_h.at[pl.ds(16 * RPT, RTAIL)],
                        rows_v.at[pl.ds(0, RTAIL)])
        pltpu.sync_copy(rows_v.at[pl.ds(0, RTAIL)],
                        out_h.at[cid, pl.ds(16 * RPT, RTAIL)])
        if with_ea:
            pltpu.sync_copy(acc_ea.at[pl.ds(16 * RPT, RTAIL)],
                            ea_v.at[pl.ds(0, RTAIL)])
            pltpu.sync_copy(ea_v.at[pl.ds(0, RTAIL)],
                            out_ea.at[cid, pl.ds(16 * RPT, RTAIL)])


def _make_sc(with_ea):
    mesh = plsc.VectorSubcoreMesh(core_axis_name="c", subcore_axis_name="s")
    out_type = [jax.ShapeDtypeStruct((2, N, D), _f32)]
    scratch = [
        pltpu.VMEM((CH,), jnp.int32),
        pltpu.VMEM((CH,), jnp.int32),
        pltpu.VMEM((CH, D), _f32),
        pltpu.SemaphoreType.DMA,
        pltpu.VMEM_SHARED((NP, D), _f32),
    ]
    if with_ea:
        out_type.append(jax.ShapeDtypeStruct((2, N, 16), _f32))
        scratch += [
            pltpu.VMEM((CH,), jnp.int32),
            pltpu.VMEM((CH, 16), _f32),
            pltpu.VMEM_SHARED((NP, 16), _f32),
        ]
    return pl.kernel(functools.partial(_sc_body, with_ea),
                     out_type=out_type, mesh=mesh, scratch_types=scratch,
                     compiler_params=pltpu.CompilerParams(
                         use_tc_tiling_on_sc=False))


_sc_l0 = _make_sc(True)
_sc_l1 = _make_sc(False)


def _proj_body(x_ref, w_ref, b_ref, o_ref):
    o_ref[...] = jnp.dot(x_ref[...], w_ref[...],
                         preferred_element_type=_f32) + b_ref[...]


def _proj(x, wT, b):
    return pl.pallas_call(
        _proj_body,
        grid=(NBLK,),
        in_specs=[pl.BlockSpec((BM, IN), lambda i: (i, 0)),
                  pl.BlockSpec((IN, D), lambda i: (0, 0)),
                  pl.BlockSpec((1, D), lambda i: (0, 0))],
        out_specs=pl.BlockSpec((BM, D), lambda i: (i, 0)),
        out_shape=jax.ShapeDtypeStruct((N, D), _f32),
    )(x, wT, b)


def _pre_body(p0, p1, h_ref, e0, e1, w1, bf, br, z_ref, s_ref, q_ref):
    a = p0[0] + p1[0] - h_ref[...]
    e = e0[0] + e1[0]
    z = (jnp.dot(a, w1[...], preferred_element_type=_f32)
         + jnp.dot(e, bf[...], preferred_element_type=_f32) + br[...])
    z_ref[...] = z

    @pl.when(pl.program_id(0) == 0)
    def _():
        s_ref[...] = jnp.zeros_like(s_ref)
        q_ref[...] = jnp.zeros_like(q_ref)

    s_ref[...] += jnp.sum(z, axis=0, keepdims=True)
    q_ref[...] += jnp.sum(z * z, axis=0, keepdims=True)


def _pre(ph, h, pe, w1l, bfull, brow):
    return pl.pallas_call(
        _pre_body,
        grid=(NBLK,),
        in_specs=[pl.BlockSpec((1, BM, D), lambda i: (0, i, 0)),
                  pl.BlockSpec((1, BM, D), lambda i: (1, i, 0)),
                  pl.BlockSpec((BM, D), lambda i: (i, 0)),
                  pl.BlockSpec((1, BM, 16), lambda i: (0, i, 0)),
                  pl.BlockSpec((1, BM, 16), lambda i: (1, i, 0)),
                  pl.BlockSpec((D, D2), lambda i: (0, 0)),
                  pl.BlockSpec((16, D2), lambda i: (0, 0)),
                  pl.BlockSpec((1, D2), lambda i: (0, 0))],
        out_specs=[pl.BlockSpec((BM, D2), lambda i: (i, 0)),
                   pl.BlockSpec((1, D2), lambda i: (0, 0)),
                   pl.BlockSpec((1, D2), lambda i: (0, 0))],
        out_shape=[jax.ShapeDtypeStruct((N, D2), _f32),
                   jax.ShapeDtypeStruct((1, D2), _f32),
                   jax.ShapeDtypeStruct((1, D2), _f32)],
    )(ph, ph, h, pe, pe, w1l, bfull, brow)


def _post_body(outer_relu, z_ref, s_ref, q_ref, g_ref, bt_ref, w2_ref,
               b2_ref, o_ref):
    mu = s_ref[...] / N
    var = q_ref[...] / N - mu * mu
    inv = lax.rsqrt(var + 1e-5)
    zn = (z_ref[...] - mu) * inv * g_ref[...] + bt_ref[...]
    zr = jnp.maximum(zn, 0.0)
    o = jnp.dot(zr, w2_ref[...], preferred_element_type=_f32) + b2_ref[...]
    if outer_relu:
        o = jnp.maximum(o, 0.0)
    o_ref[...] = o


def _post(z, s, q, g, bt, w2T, b2, outer_relu):
    return pl.pallas_call(
        functools.partial(_post_body, outer_relu),
        grid=(NBLK,),
        in_specs=[pl.BlockSpec((BM, D2), lambda i: (i, 0)),
                  pl.BlockSpec((1, D2), lambda i: (0, 0)),
                  pl.BlockSpec((1, D2), lambda i: (0, 0)),
                  pl.BlockSpec((1, D2), lambda i: (0, 0)),
                  pl.BlockSpec((1, D2), lambda i: (0, 0)),
                  pl.BlockSpec((D2, D), lambda i: (0, 0)),
                  pl.BlockSpec((1, D), lambda i: (0, 0))],
        out_specs=pl.BlockSpec((BM, D), lambda i: (i, 0)),
        out_shape=jax.ShapeDtypeStruct((N, D), _f32),
    )(z, s, q, g, bt, w2T, b2)


def _layer(h, ph, pe, W1, b1, We, be, g, bt, W2, b2, outer_relu):
    w1l = W1[:, :D].T
    w1rT = W1[:, D:].T
    b7 = We.T @ w1rT
    c = be @ w1rT
    bfull = jnp.zeros((16, D2), _f32).at[:7].set(b7).at[7].set(c)
    brow = (b1 + b7[6] + c).reshape(1, D2)
    z, s, q = _pre(ph, h, pe, w1l, bfull, brow)
    return _post(z, s, q, g.reshape(1, D2), bt.reshape(1, D2), W2.T,
                 b2.reshape(1, D), outer_relu)


def kernel(x, edge_index, edge_attr, Win, bin_, We0, be0, W1_0, b1_0, g0,
           bt0, W2_0, b2_0, We1, be1, W1_1, b1_1, g1, bt1, W2_1, b2_1):
    src = edge_index[0].astype(jnp.int32)
    dst = edge_index[1].astype(jnp.int32)
    src_r = jnp.pad(src, (0, EPAD - E)).reshape(TCH, CH)
    dst_r = jnp.pad(dst, (0, EPAD - E),
                    constant_values=DUMMY).reshape(TCH, CH)
    ea = jnp.concatenate(
        [edge_attr.astype(_f32), jnp.ones((E, 1), _f32),
         jnp.zeros((E, 8), _f32)], axis=1)
    ea_r = jnp.pad(ea, ((0, EPAD - E), (0, 0)))
    eid_r = jnp.arange(EPAD, dtype=jnp.int32).reshape(TCH, CH)

    h0 = _proj(x, Win.T, bin_.reshape(1, D))

    ph0, pe = _sc_l0(h0, src_r, dst_r, eid_r, ea_r)
    h1 = _layer(h0, ph0, pe, W1_0, b1_0, We0, be0, g0, bt0, W2_0, b2_0,
                outer_relu=True)

    (ph1,) = _sc_l1(h1, src_r, dst_r)
    return _layer(h1, ph1, pe, W1_1, b1_1, We1, be1, g1, bt1, W2_1, b2_1,
                  outer_relu=False)


# speedup vs baseline: 1.1957x; 1.0588x over previous
import functools

import jax
import jax.numpy as jnp
from jax import lax
from jax.experimental import pallas as pl
from jax.experimental.pallas import tpu as pltpu
from jax.experimental.pallas import tpu_sc as plsc

N, E, D, IN = 10000, 320000, 128, 1024
D2 = 2 * D
NTILES = 32
CH = 128
TCH = 2528
NCH0 = 92
NCH1 = 66
NCH0_L1 = 86
NCH1_L1 = 72
EPAD = TCH * CH
DUMMY = N
NP = N + 16
RPT = 624
RC = 104
RTAIL = N - 16 * RPT
BM = 400
NBLK = N // BM

_f32 = jnp.float32


def _sc_body(with_ea, *refs):
    if with_ea:
        (h_hbm, src_hbm, dst_hbm, eid_hbm, ea_hbm, out_h, out_ea,
         src_v, dst_v, rows_v, sem, acc_h, eid_v, ea_v, acc_ea) = refs
    else:
        (h_hbm, src_hbm, dst_hbm, out_h,
         src_v, dst_v, rows_v, sem, acc_h) = refs

    cid = lax.axis_index("c")
    sid = lax.axis_index("s")
    n0, n1 = (NCH0, NCH1) if with_ea else (NCH0_L1, NCH1_L1)
    base = jnp.where(cid == 0, sid * n0, 16 * n0 + sid * n1)
    nw = jnp.where(cid == 0, n0, n1)

    for k in range(RPT // RC):
        r0 = sid * RPT + k * RC
        pltpu.sync_copy(h_hbm.at[pl.ds(r0, RC)], rows_v.at[pl.ds(0, RC)])
        pltpu.sync_copy(rows_v.at[pl.ds(0, RC)], acc_h.at[pl.ds(r0, RC)])

    @pl.when(sid == 15)
    def _():
        pltpu.sync_copy(h_hbm.at[pl.ds(16 * RPT, RTAIL)],
                        rows_v.at[pl.ds(0, RTAIL)])
        pltpu.sync_copy(rows_v.at[pl.ds(0, RTAIL)],
                        acc_h.at[pl.ds(16 * RPT, RTAIL)])

    if with_ea:
        def zrow(j, carry):
            ea_v[j, :] = jnp.zeros((16,), _f32)
            return carry

        lax.fori_loop(0, CH, zrow, 0)
        for k in range(RPT // RC):
            r0 = sid * RPT + k * RC
            pltpu.sync_copy(ea_v.at[pl.ds(0, RC)], acc_ea.at[pl.ds(r0, RC)])

        @pl.when(sid == 15)
        def _():
            pltpu.sync_copy(ea_v.at[pl.ds(0, RTAIL)],
                            acc_ea.at[pl.ds(16 * RPT, RTAIL)])

    plsc.subcore_barrier()

    def chunk(j, carry):
        q = base + j
        pltpu.sync_copy(src_hbm.at[q], src_v)
        pltpu.sync_copy(dst_hbm.at[q], dst_v)
        pltpu.async_copy(h_hbm.at[src_v], rows_v, sem).wait()
        pltpu.sync_copy(rows_v, acc_h.at[dst_v], add=True)
        if with_ea:
            pltpu.sync_copy(eid_hbm.at[q], eid_v)
            pltpu.async_copy(ea_hbm.at[eid_v], ea_v, sem).wait()
            pltpu.sync_copy(ea_v, acc_ea.at[dst_v], add=True)
        return carry

    lax.fori_loop(0, nw, chunk, 0)
    plsc.subcore_barrier()

    for k in range(RPT // RC):
        r0 = sid * RPT + k * RC
        pltpu.sync_copy(acc_h.at[pl.ds(r0, RC)], rows_v.at[pl.ds(0, RC)])
        pltpu.sync_copy(rows_v.at[pl.ds(0, RC)],
                        out_h.at[cid, pl.ds(r0, RC)])
        if with_ea:
            pltpu.sync_copy(acc_ea.at[pl.ds(r0, RC)], ea_v.at[pl.ds(0, RC)])
            pltpu.sync_copy(ea_v.at[pl.ds(0, RC)],
                            out_ea.at[cid, pl.ds(r0, RC)])

    @pl.when(sid == 15)
    def _():
        pltpu.sync_copy(acc_h.at[pl.ds(16 * RPT, RTAIL)],
                        rows_v.at[pl.ds(0, RTAIL)])
        pltpu.sync_copy(rows_v.at[pl.ds(0, RTAIL)],
                        out_h.at[cid, pl.ds(16 * RPT, RTAIL)])
        if with_ea:
            pltpu.sync_copy(acc_ea.at[pl.ds(16 * RPT, RTAIL)],
                            ea_v.at[pl.ds(0, RTAIL)])
            pltpu.sync_copy(ea_v.at[pl.ds(0, RTAIL)],
                            out_ea.at[cid, pl.ds(16 * RPT, RTAIL)])


def _make_sc(with_ea):
    mesh = plsc.VectorSubcoreMesh(core_axis_name="c", subcore_axis_name="s")
    out_type = [jax.ShapeDtypeStruct((2, N, D), _f32)]
    scratch = [
        pltpu.VMEM((CH,), jnp.int32),
        pltpu.VMEM((CH,), jnp.int32),
        pltpu.VMEM((CH, D), _f32),
        pltpu.SemaphoreType.DMA,
        pltpu.VMEM_SHARED((NP, D), _f32),
    ]
    if with_ea:
        out_type.append(jax.ShapeDtypeStruct((2, N, 16), _f32))
        scratch += [
            pltpu.VMEM((CH,), jnp.int32),
            pltpu.VMEM((CH, 16), _f32),
            pltpu.VMEM_SHARED((NP, 16), _f32),
        ]
    return pl.kernel(functools.partial(_sc_body, with_ea),
                     out_type=out_type, mesh=mesh, scratch_types=scratch,
                     compiler_params=pltpu.CompilerParams(
                         use_tc_tiling_on_sc=False))


_sc_l0 = _make_sc(True)
_sc_l1 = _make_sc(False)


def _proj_body(x_ref, w_ref, b_ref, o_ref):
    o_ref[...] = jnp.dot(x_ref[...], w_ref[...],
                         preferred_element_type=_f32) + b_ref[...]


def _proj(x, wT, b):
    return pl.pallas_call(
        _proj_body,
        grid=(NBLK,),
        in_specs=[pl.BlockSpec((BM, IN), lambda i: (i, 0)),
                  pl.BlockSpec((IN, D), lambda i: (0, 0)),
                  pl.BlockSpec((1, D), lambda i: (0, 0))],
        out_specs=pl.BlockSpec((BM, D), lambda i: (i, 0)),
        out_shape=jax.ShapeDtypeStruct((N, D), _f32),
    )(x, wT, b)


def _pre_body(p0, p1, h_ref, e0, e1, w1, bf, br, z_ref, s_ref, q_ref):
    a = p0[0] + p1[0] - h_ref[...]
    e = e0[0] + e1[0]
    z = (jnp.dot(a, w1[...], preferred_element_type=_f32)
         + jnp.dot(e, bf[...], preferred_element_type=_f32) + br[...])
    z_ref[...] = z

    @pl.when(pl.program_id(0) == 0)
    def _():
        s_ref[...] = jnp.zeros_like(s_ref)
        q_ref[...] = jnp.zeros_like(q_ref)

    s_ref[...] += jnp.sum(z, axis=0, keepdims=True)
    q_ref[...] += jnp.sum(z * z, axis=0, keepdims=True)


def _pre(ph, h, pe, w1l, bfull, brow):
    return pl.pallas_call(
        _pre_body,
        grid=(NBLK,),
        in_specs=[pl.BlockSpec((1, BM, D), lambda i: (0, i, 0)),
                  pl.BlockSpec((1, BM, D), lambda i: (1, i, 0)),
                  pl.BlockSpec((BM, D), lambda i: (i, 0)),
                  pl.BlockSpec((1, BM, 16), lambda i: (0, i, 0)),
                  pl.BlockSpec((1, BM, 16), lambda i: (1, i, 0)),
                  pl.BlockSpec((D, D2), lambda i: (0, 0)),
                  pl.BlockSpec((16, D2), lambda i: (0, 0)),
                  pl.BlockSpec((1, D2), lambda i: (0, 0))],
        out_specs=[pl.BlockSpec((BM, D2), lambda i: (i, 0)),
                   pl.BlockSpec((1, D2), lambda i: (0, 0)),
                   pl.BlockSpec((1, D2), lambda i: (0, 0))],
        out_shape=[jax.ShapeDtypeStruct((N, D2), _f32),
                   jax.ShapeDtypeStruct((1, D2), _f32),
                   jax.ShapeDtypeStruct((1, D2), _f32)],
    )(ph, ph, h, pe, pe, w1l, bfull, brow)


def _post_body(outer_relu, z_ref, s_ref, q_ref, g_ref, bt_ref, w2_ref,
               b2_ref, o_ref):
    mu = s_ref[...] / N
    var = q_ref[...] / N - mu * mu
    inv = lax.rsqrt(var + 1e-5)
    zn = (z_ref[...] - mu) * inv * g_ref[...] + bt_ref[...]
    zr = jnp.maximum(zn, 0.0)
    o = jnp.dot(zr, w2_ref[...], preferred_element_type=_f32) + b2_ref[...]
    if outer_relu:
        o = jnp.maximum(o, 0.0)
    o_ref[...] = o


def _post(z, s, q, g, bt, w2T, b2, outer_relu):
    return pl.pallas_call(
        functools.partial(_post_body, outer_relu),
        grid=(NBLK,),
        in_specs=[pl.BlockSpec((BM, D2), lambda i: (i, 0)),
                  pl.BlockSpec((1, D2), lambda i: (0, 0)),
                  pl.BlockSpec((1, D2), lambda i: (0, 0)),
                  pl.BlockSpec((1, D2), lambda i: (0, 0)),
                  pl.BlockSpec((1, D2), lambda i: (0, 0)),
                  pl.BlockSpec((D2, D), lambda i: (0, 0)),
                  pl.BlockSpec((1, D), lambda i: (0, 0))],
        out_specs=pl.BlockSpec((BM, D), lambda i: (i, 0)),
        out_shape=jax.ShapeDtypeStruct((N, D), _f32),
    )(z, s, q, g, bt, w2T, b2)


def _layer(h, ph, pe, W1, b1, We, be, g, bt, W2, b2, outer_relu):
    w1l = W1[:, :D].T
    w1rT = W1[:, D:].T
    b7 = We.T @ w1rT
    c = be @ w1rT
    bfull = jnp.zeros((16, D2), _f32).at[:7].set(b7).at[7].set(c)
    brow = (b1 + b7[6] + c).reshape(1, D2)
    z, s, q = _pre(ph, h, pe, w1l, bfull, brow)
    return _post(z, s, q, g.reshape(1, D2), bt.reshape(1, D2), W2.T,
                 b2.reshape(1, D), outer_relu)


def kernel(x, edge_index, edge_attr, Win, bin_, We0, be0, W1_0, b1_0, g0,
           bt0, W2_0, b2_0, We1, be1, W1_1, b1_1, g1, bt1, W2_1, b2_1):
    src = edge_index[0].astype(jnp.int32)
    dst = edge_index[1].astype(jnp.int32)
    src_r = jnp.pad(src, (0, EPAD - E)).reshape(TCH, CH)
    dst_r = jnp.pad(dst, (0, EPAD - E),
                    constant_values=DUMMY).reshape(TCH, CH)
    ea_r = jnp.concatenate(
        [edge_attr.astype(_f32), jnp.ones((E, 1), _f32),
         jnp.zeros((E, 8), _f32)], axis=1)
    eid_r = jnp.minimum(jnp.arange(EPAD, dtype=jnp.int32),
                        E - 1).reshape(TCH, CH)

    h0 = _proj(x, Win.T, bin_.reshape(1, D))

    ph0, pe = _sc_l0(h0, src_r, dst_r, eid_r, ea_r)
    h1 = _layer(h0, ph0, pe, W1_0, b1_0, We0, be0, g0, bt0, W2_0, b2_0,
                outer_relu=True)

    (ph1,) = _sc_l1(h1, src_r, dst_r)
    return _layer(h1, ph1, pe, W1_1, b1_1, We1, be1, g1, bt1, W2_1, b2_1,
                  outer_relu=False)


# speedup vs baseline: 1.2090x; 1.0111x over previous
import functools

import jax
import jax.numpy as jnp
from jax import lax
from jax.experimental import pallas as pl
from jax.experimental.pallas import tpu as pltpu
from jax.experimental.pallas import tpu_sc as plsc

N, E, D, IN = 10000, 320000, 128, 1024
D2 = 2 * D
NTILES = 32
CH = 128
TCH = 2528
NCH0 = 88
NCH1 = 70
NCH0_L1 = 92
NCH1_L1 = 66
EPAD = TCH * CH
DUMMY = N
NP = N + 16
RPT = 624
RC = 104
RTAIL = N - 16 * RPT
BM = 400
NBLK = N // BM

_f32 = jnp.float32


def _sc_body(with_ea, *refs):
    if with_ea:
        (h_hbm, src_hbm, dst_hbm, eid_hbm, ea_hbm, out_h, out_ea,
         src_v, dst_v, rows_v, sem, acc_h, eid_v, ea_v, acc_ea) = refs
    else:
        (h_hbm, src_hbm, dst_hbm, out_h,
         src_v, dst_v, rows_v, sem, acc_h) = refs

    cid = lax.axis_index("c")
    sid = lax.axis_index("s")
    n0, n1 = (NCH0, NCH1) if with_ea else (NCH0_L1, NCH1_L1)
    base = jnp.where(cid == 0, sid * n0, 16 * n0 + sid * n1)
    nw = jnp.where(cid == 0, n0, n1)

    for k in range(RPT // RC):
        r0 = sid * RPT + k * RC
        pltpu.sync_copy(h_hbm.at[pl.ds(r0, RC)], rows_v.at[pl.ds(0, RC)])
        pltpu.sync_copy(rows_v.at[pl.ds(0, RC)], acc_h.at[pl.ds(r0, RC)])

    @pl.when(sid == 15)
    def _():
        pltpu.sync_copy(h_hbm.at[pl.ds(16 * RPT, RTAIL)],
                        rows_v.at[pl.ds(0, RTAIL)])
        pltpu.sync_copy(rows_v.at[pl.ds(0, RTAIL)],
                        acc_h.at[pl.ds(16 * RPT, RTAIL)])

    if with_ea:
        def zrow(j, carry):
            ea_v[j, :] = jnp.zeros((16,), _f32)
            return carry

        lax.fori_loop(0, CH, zrow, 0)
        for k in range(RPT // RC):
            r0 = sid * RPT + k * RC
            pltpu.sync_copy(ea_v.at[pl.ds(0, RC)], acc_ea.at[pl.ds(r0, RC)])

        @pl.when(sid == 15)
        def _():
            pltpu.sync_copy(ea_v.at[pl.ds(0, RTAIL)],
                            acc_ea.at[pl.ds(16 * RPT, RTAIL)])

    plsc.subcore_barrier()

    def chunk(j, carry):
        q = base + j
        pltpu.sync_copy(src_hbm.at[q], src_v)
        pltpu.sync_copy(dst_hbm.at[q], dst_v)
        pltpu.async_copy(h_hbm.at[src_v], rows_v, sem).wait()
        pltpu.sync_copy(rows_v, acc_h.at[dst_v], add=True)
        if with_ea:
            pltpu.sync_copy(eid_hbm.at[q], eid_v)
            pltpu.async_copy(ea_hbm.at[eid_v], ea_v, sem).wait()
            pltpu.sync_copy(ea_v, acc_ea.at[dst_v], add=True)
        return carry

    lax.fori_loop(0, nw, chunk, 0)
    plsc.subcore_barrier()

    for k in range(RPT // RC):
        r0 = sid * RPT + k * RC
        pltpu.sync_copy(acc_h.at[pl.ds(r0, RC)], rows_v.at[pl.ds(0, RC)])
        pltpu.sync_copy(rows_v.at[pl.ds(0, RC)],
                        out_h.at[cid, pl.ds(r0, RC)])
        if with_ea:
            pltpu.sync_copy(acc_ea.at[pl.ds(r0, RC)], ea_v.at[pl.ds(0, RC)])
            pltpu.sync_copy(ea_v.at[pl.ds(0, RC)],
                            out_ea.at[cid, pl.ds(r0, RC)])

    @pl.when(sid == 15)
    def _():
        pltpu.sync_copy(acc_h.at[pl.ds(16 * RPT, RTAIL)],
                        rows_v.at[pl.ds(0, RTAIL)])
        pltpu.sync_copy(rows_v.at[pl.ds(0, RTAIL)],
                        out_h.at[cid, pl.ds(16 * RPT, RTAIL)])
        if with_ea:
            pltpu.sync_copy(acc_ea.at[pl.ds(16 * RPT, RTAIL)],
                            ea_v.at[pl.ds(0, RTAIL)])
            pltpu.sync_copy(ea_v.at[pl.ds(0, RTAIL)],
                            out_ea.at[cid, pl.ds(16 * RPT, RTAIL)])


def _make_sc(with_ea):
    mesh = plsc.VectorSubcoreMesh(core_axis_name="c", subcore_axis_name="s")
    out_type = [jax.ShapeDtypeStruct((2, N, D), _f32)]
    scratch = [
        pltpu.VMEM((CH,), jnp.int32),
        pltpu.VMEM((CH,), jnp.int32),
        pltpu.VMEM((CH, D), _f32),
        pltpu.SemaphoreType.DMA,
        pltpu.VMEM_SHARED((NP, D), _f32),
    ]
    if with_ea:
        out_type.append(jax.ShapeDtypeStruct((2, N, 16), _f32))
        scratch += [
            pltpu.VMEM((CH,), jnp.int32),
            pltpu.VMEM((CH, 16), _f32),
            pltpu.VMEM_SHARED((NP, 16), _f32),
        ]
    return pl.kernel(functools.partial(_sc_body, with_ea),
                     out_type=out_type, mesh=mesh, scratch_types=scratch,
                     compiler_params=pltpu.CompilerParams(
                         use_tc_tiling_on_sc=False))


_sc_l0 = _make_sc(True)
_sc_l1 = _make_sc(False)


def _proj_body(x_ref, w_ref, b_ref, o_ref):
    o_ref[...] = jnp.dot(x_ref[...], w_ref[...],
                         preferred_element_type=_f32) + b_ref[...]


def _proj(x, wT, b):
    return pl.pallas_call(
        _proj_body,
        grid=(NBLK,),
        in_specs=[pl.BlockSpec((BM, IN), lambda i: (i, 0)),
                  pl.BlockSpec((IN, D), lambda i: (0, 0)),
                  pl.BlockSpec((1, D), lambda i: (0, 0))],
        out_specs=pl.BlockSpec((BM, D), lambda i: (i, 0)),
        out_shape=jax.ShapeDtypeStruct((N, D), _f32),
    )(x, wT, b)


def _pre_body(p0, p1, h_ref, e0, e1, w1, bf, br, z_ref, s_ref, q_ref):
    a = p0[0] + p1[0] - h_ref[...]
    e = e0[0] + e1[0]
    z = (jnp.dot(a, w1[...], preferred_element_type=_f32)
         + jnp.dot(e, bf[...], preferred_element_type=_f32) + br[...])
    z_ref[...] = z

    @pl.when(pl.program_id(0) == 0)
    def _():
        s_ref[...] = jnp.zeros_like(s_ref)
        q_ref[...] = jnp.zeros_like(q_ref)

    s_ref[...] += jnp.sum(z, axis=0, keepdims=True)
    q_ref[...] += jnp.sum(z * z, axis=0, keepdims=True)


def _pre(ph, h, pe, w1l, bfull, brow):
    return pl.pallas_call(
        _pre_body,
        grid=(NBLK,),
        in_specs=[pl.BlockSpec((1, BM, D), lambda i: (0, i, 0)),
                  pl.BlockSpec((1, BM, D), lambda i: (1, i, 0)),
                  pl.BlockSpec((BM, D), lambda i: (i, 0)),
                  pl.BlockSpec((1, BM, 16), lambda i: (0, i, 0)),
                  pl.BlockSpec((1, BM, 16), lambda i: (1, i, 0)),
                  pl.BlockSpec((D, D2), lambda i: (0, 0)),
                  pl.BlockSpec((16, D2), lambda i: (0, 0)),
                  pl.BlockSpec((1, D2), lambda i: (0, 0))],
        out_specs=[pl.BlockSpec((BM, D2), lambda i: (i, 0)),
                   pl.BlockSpec((1, D2), lambda i: (0, 0)),
                   pl.BlockSpec((1, D2), lambda i: (0, 0))],
        out_shape=[jax.ShapeDtypeStruct((N, D2), _f32),
                   jax.ShapeDtypeStruct((1, D2), _f32),
                   jax.ShapeDtypeStruct((1, D2), _f32)],
    )(ph, ph, h, pe, pe, w1l, bfull, brow)


def _post_body(outer_relu, z_ref, s_ref, q_ref, g_ref, bt_ref, w2_ref,
               b2_ref, o_ref):
    mu = s_ref[...] / N
    var = q_ref[...] / N - mu * mu
    inv = lax.rsqrt(var + 1e-5)
    zn = (z_ref[...] - mu) * inv * g_ref[...] + bt_ref[...]
    zr = jnp.maximum(zn, 0.0)
    o = jnp.dot(zr, w2_ref[...], preferred_element_type=_f32) + b2_ref[...]
    if outer_relu:
        o = jnp.maximum(o, 0.0)
    o_ref[...] = o


def _post(z, s, q, g, bt, w2T, b2, outer_relu):
    return pl.pallas_call(
        functools.partial(_post_body, outer_relu),
        grid=(NBLK,),
        in_specs=[pl.BlockSpec((BM, D2), lambda i: (i, 0)),
                  pl.BlockSpec((1, D2), lambda i: (0, 0)),
                  pl.BlockSpec((1, D2), lambda i: (0, 0)),
                  pl.BlockSpec((1, D2), lambda i: (0, 0)),
                  pl.BlockSpec((1, D2), lambda i: (0, 0)),
                  pl.BlockSpec((D2, D), lambda i: (0, 0)),
                  pl.BlockSpec((1, D), lambda i: (0, 0))],
        out_specs=pl.BlockSpec((BM, D), lambda i: (i, 0)),
        out_shape=jax.ShapeDtypeStruct((N, D), _f32),
    )(z, s, q, g, bt, w2T, b2)


def _layer(h, ph, pe, W1, b1, We, be, g, bt, W2, b2, outer_relu):
    w1l = W1[:, :D].T
    w1rT = W1[:, D:].T
    b7 = We.T @ w1rT
    c = be @ w1rT
    bfull = jnp.zeros((16, D2), _f32).at[:7].set(b7).at[7].set(c)
    brow = (b1 + b7[6] + c).reshape(1, D2)
    z, s, q = _pre(ph, h, pe, w1l, bfull, brow)
    return _post(z, s, q, g.reshape(1, D2), bt.reshape(1, D2), W2.T,
                 b2.reshape(1, D), outer_relu)


def kernel(x, edge_index, edge_attr, Win, bin_, We0, be0, W1_0, b1_0, g0,
           bt0, W2_0, b2_0, We1, be1, W1_1, b1_1, g1, bt1, W2_1, b2_1):
    src = edge_index[0].astype(jnp.int32)
    dst = edge_index[1].astype(jnp.int32)
    src_r = jnp.pad(src, (0, EPAD - E)).reshape(TCH, CH)
    dst_r = jnp.pad(dst, (0, EPAD - E),
                    constant_values=DUMMY).reshape(TCH, CH)
    ea_r = jnp.concatenate(
        [edge_attr.astype(_f32), jnp.ones((E, 1), _f32),
         jnp.zeros((E, 8), _f32)], axis=1)
    eid_r = jnp.minimum(jnp.arange(EPAD, dtype=jnp.int32),
                        E - 1).reshape(TCH, CH)

    h0 = _proj(x, Win.T, bin_.reshape(1, D))

    ph0, pe = _sc_l0(h0, src_r, dst_r, eid_r, ea_r)
    h1 = _layer(h0, ph0, pe, W1_0, b1_0, We0, be0, g0, bt0, W2_0, b2_0,
                outer_relu=True)

    (ph1,) = _sc_l1(h1, src_r, dst_r)
    return _layer(h1, ph1, pe, W1_1, b1_1, We1, be1, g1, bt1, W2_1, b2_1,
                  outer_relu=False)
